# Initial kernel scaffold; baseline (speedup 1.0000x reference)
import jax
import jax.numpy as jnp
from jax.experimental import pallas as pl


def kernel(inputs, mask, W_ih, W_hh, b_ih, b_hh):
    raise NotImplementedError("write your pallas kernel here")


# speedup vs baseline: 7.8707x; 7.8707x over previous
import jax
import jax.numpy as jnp
from jax.experimental import pallas as pl
from jax.experimental.pallas import tpu as pltpu

B, S, D, H = 16, 512, 256, 256
TS = 64


def _lstm_kernel(x_ref, m_ref, wih_ref, whh_ref, b_ref,
                 out_ref, hN_ref, cN_ref, h_ref, c_ref, xpre_ref):
    @pl.when(pl.program_id(0) == 0)
    def _init():
        h_ref[...] = jnp.zeros_like(h_ref)
        c_ref[...] = jnp.zeros_like(c_ref)

    x = x_ref[...].reshape(TS * B, D)
    xpre = jnp.dot(x, wih_ref[...], preferred_element_type=jnp.float32)
    xpre_ref[...] = xpre.reshape(TS, B, 4 * H) + b_ref[...]

    def step(t, carry):
        h, c = carry
        gates = xpre_ref[t] + jnp.dot(h, whh_ref[...],
                                      preferred_element_type=jnp.float32)
        i = jax.nn.sigmoid(gates[:, 0:H])
        f = jax.nn.sigmoid(gates[:, H:2 * H])
        g = jnp.tanh(gates[:, 2 * H:3 * H])
        o = jax.nn.sigmoid(gates[:, 3 * H:4 * H])
        c_new = f * c + i * g
        h_new = o * jnp.tanh(c_new)
        m2 = m_ref[t]
        out_ref[t] = h_new * m2
        h = m2 * h_new + (1.0 - m2) * h
        c = m2 * c_new + (1.0 - m2) * c
        return h, c

    h, c = jax.lax.fori_loop(0, TS, step, (h_ref[...], c_ref[...]))
    h_ref[...] = h
    c_ref[...] = c
    hN_ref[...] = h
    cN_ref[...] = c


def kernel(inputs, mask, W_ih, W_hh, b_ih, b_hh):
    x_tm = jnp.swapaxes(inputs, 0, 1)
    m_tm = jnp.swapaxes(mask, 0, 1).astype(inputs.dtype)[..., None]
    wih_t = W_ih.T
    whh_t = W_hh.T
    b = (b_ih + b_hh)[None, None, :]

    grid = (S // TS,)
    out, hN, cN = pl.pallas_call(
        _lstm_kernel,
        grid=grid,
        in_specs=[
            pl.BlockSpec((TS, B, D), lambda i: (i, 0, 0)),
            pl.BlockSpec((TS, B, 1), lambda i: (i, 0, 0)),
            pl.BlockSpec((D, 4 * H), lambda i: (0, 0)),
            pl.BlockSpec((H, 4 * H), lambda i: (0, 0)),
            pl.BlockSpec((1, 1, 4 * H), lambda i: (0, 0, 0)),
        ],
        out_specs=[
            pl.BlockSpec((TS, B, H), lambda i: (i, 0, 0)),
            pl.BlockSpec((B, H), lambda i: (0, 0)),
            pl.BlockSpec((B, H), lambda i: (0, 0)),
        ],
        out_shape=[
            jax.ShapeDtypeStruct((S, B, H), jnp.float32),
            jax.ShapeDtypeStruct((B, H), jnp.float32),
            jax.ShapeDtypeStruct((B, H), jnp.float32),
        ],
        scratch_shapes=[
            pltpu.VMEM((B, H), jnp.float32),
            pltpu.VMEM((B, H), jnp.float32),
            pltpu.VMEM((TS, B, 4 * H), jnp.float32),
        ],
    )(x_tm, m_tm, wih_t, whh_t, b)

    outputs = jnp.swapaxes(out, 0, 1)
    return outputs, hN[None, :, :], cN[None, :, :]


# speedup vs baseline: 8.6849x; 1.1034x over previous
import jax
import jax.numpy as jnp
from jax.experimental import pallas as pl
from jax.experimental.pallas import tpu as pltpu

B, S, D, H = 16, 512, 256, 256
TS = 64


def _lstm_kernel(x_ref, m_ref, wih_ref, whh_ref, b_ref,
                 out_ref, hN_ref, cN_ref, h_ref, c_ref, xpre_ref):
    @pl.when(pl.program_id(0) == 0)
    def _init():
        h_ref[...] = jnp.zeros_like(h_ref)
        c_ref[...] = jnp.zeros_like(c_ref)

    x = x_ref[...].reshape(TS * B, D)
    xpre = jnp.dot(x, wih_ref[...], preferred_element_type=jnp.float32)
    xpre_ref[...] = xpre.reshape(TS, B, 4 * H) + b_ref[...]

    def step(t, carry):
        h, c = carry
        gates = xpre_ref[t] + jnp.dot(h, whh_ref[...],
                                      preferred_element_type=jnp.float32)
        i = jax.nn.sigmoid(gates[:, 0:H])
        f = jax.nn.sigmoid(gates[:, H:2 * H])
        g = jnp.tanh(gates[:, 2 * H:3 * H])
        o = jax.nn.sigmoid(gates[:, 3 * H:4 * H])
        c_new = f * c + i * g
        h_new = o * jnp.tanh(c_new)
        m2 = m_ref[t]
        out_ref[t] = h_new * m2
        h = m2 * h_new + (1.0 - m2) * h
        c = m2 * c_new + (1.0 - m2) * c
        return h, c

    h, c = jax.lax.fori_loop(0, TS, step, (h_ref[...], c_ref[...]),
                             unroll=4)
    h_ref[...] = h
    c_ref[...] = c
    hN_ref[...] = h
    cN_ref[...] = c


def kernel(inputs, mask, W_ih, W_hh, b_ih, b_hh):
    x_tm = jnp.swapaxes(inputs, 0, 1)
    m_tm = jnp.swapaxes(mask, 0, 1).astype(inputs.dtype)[..., None]
    wih_t = W_ih.T
    whh_t = W_hh.T
    b = (b_ih + b_hh)[None, None, :]

    grid = (S // TS,)
    out, hN, cN = pl.pallas_call(
        _lstm_kernel,
        grid=grid,
        in_specs=[
            pl.BlockSpec((TS, B, D), lambda i: (i, 0, 0)),
            pl.BlockSpec((TS, B, 1), lambda i: (i, 0, 0)),
            pl.BlockSpec((D, 4 * H), lambda i: (0, 0)),
            pl.BlockSpec((H, 4 * H), lambda i: (0, 0)),
            pl.BlockSpec((1, 1, 4 * H), lambda i: (0, 0, 0)),
        ],
        out_specs=[
            pl.BlockSpec((TS, B, H), lambda i: (i, 0, 0)),
            pl.BlockSpec((B, H), lambda i: (0, 0)),
            pl.BlockSpec((B, H), lambda i: (0, 0)),
        ],
        out_shape=[
            jax.ShapeDtypeStruct((S, B, H), jnp.float32),
            jax.ShapeDtypeStruct((B, H), jnp.float32),
            jax.ShapeDtypeStruct((B, H), jnp.float32),
        ],
        scratch_shapes=[
            pltpu.VMEM((B, H), jnp.float32),
            pltpu.VMEM((B, H), jnp.float32),
            pltpu.VMEM((TS, B, 4 * H), jnp.float32),
        ],
    )(x_tm, m_tm, wih_t, whh_t, b)

    outputs = jnp.swapaxes(out, 0, 1)
    return outputs, hN[None, :, :], cN[None, :, :]


# speedup vs baseline: 8.8087x; 1.0143x over previous
import jax
import jax.numpy as jnp
from jax.experimental import pallas as pl
from jax.experimental.pallas import tpu as pltpu

B, S, D, H = 16, 512, 256, 256
TS = 64


def _lstm_kernel(x_ref, m_ref, wih_ref, whh_ref, b_ref,
                 out_ref, hN_ref, cN_ref, h_ref, c_ref, xpre_ref):
    @pl.when(pl.program_id(0) == 0)
    def _init():
        h_ref[...] = jnp.zeros_like(h_ref)
        c_ref[...] = jnp.zeros_like(c_ref)

    x = x_ref[...].reshape(TS * B, D)
    xpre = jnp.dot(x, wih_ref[...], preferred_element_type=jnp.float32)
    xpre_ref[...] = xpre.reshape(TS, B, 4 * H) + b_ref[...]

    def step(t, carry):
        h, c = carry
        gates = xpre_ref[t] + jnp.dot(h, whh_ref[...],
                                      preferred_element_type=jnp.float32)
        i = jax.nn.sigmoid(gates[:, 0:H])
        f = jax.nn.sigmoid(gates[:, H:2 * H])
        g = jnp.tanh(gates[:, 2 * H:3 * H])
        o = jax.nn.sigmoid(gates[:, 3 * H:4 * H])
        c_new = f * c + i * g
        h_new = o * jnp.tanh(c_new)
        m2 = m_ref[t]
        out_ref[t] = h_new * m2
        h = m2 * h_new + (1.0 - m2) * h
        c = m2 * c_new + (1.0 - m2) * c
        return h, c

    h, c = jax.lax.fori_loop(0, TS, step, (h_ref[...], c_ref[...]),
                             unroll=8)
    h_ref[...] = h
    c_ref[...] = c
    hN_ref[...] = h
    cN_ref[...] = c


def kernel(inputs, mask, W_ih, W_hh, b_ih, b_hh):
    x_tm = jnp.swapaxes(inputs, 0, 1)
    m_tm = jnp.swapaxes(mask, 0, 1).astype(inputs.dtype)[..., None]
    wih_t = W_ih.T
    whh_t = W_hh.T
    b = (b_ih + b_hh)[None, None, :]

    grid = (S // TS,)
    out, hN, cN = pl.pallas_call(
        _lstm_kernel,
        grid=grid,
        in_specs=[
            pl.BlockSpec((TS, B, D), lambda i: (i, 0, 0)),
            pl.BlockSpec((TS, B, 1), lambda i: (i, 0, 0)),
            pl.BlockSpec((D, 4 * H), lambda i: (0, 0)),
            pl.BlockSpec((H, 4 * H), lambda i: (0, 0)),
            pl.BlockSpec((1, 1, 4 * H), lambda i: (0, 0, 0)),
        ],
        out_specs=[
            pl.BlockSpec((TS, B, H), lambda i: (i, 0, 0)),
            pl.BlockSpec((B, H), lambda i: (0, 0)),
            pl.BlockSpec((B, H), lambda i: (0, 0)),
        ],
        out_shape=[
            jax.ShapeDtypeStruct((S, B, H), jnp.float32),
            jax.ShapeDtypeStruct((B, H), jnp.float32),
            jax.ShapeDtypeStruct((B, H), jnp.float32),
        ],
        scratch_shapes=[
            pltpu.VMEM((B, H), jnp.float32),
            pltpu.VMEM((B, H), jnp.float32),
            pltpu.VMEM((TS, B, 4 * H), jnp.float32),
        ],
    )(x_tm, m_tm, wih_t, whh_t, b)

    outputs = jnp.swapaxes(out, 0, 1)
    return outputs, hN[None, :, :], cN[None, :, :]
